# initial kernel scaffold (unmeasured)
import jax
import jax.numpy as jnp
from jax import lax
from jax.experimental import pallas as pl
from jax.experimental.pallas import tpu as pltpu

N_DEV = 8
B = 128
D = 128
ROUNDS = 3
ROWS_PER = B // N_DEV


def kernel(x, Win0, Wout0, Win1, Wout1, Win2, Wout2):
    def body(
        x_ref, win0_ref, wout0_ref, win1_ref, wout1_ref, win2_ref, wout2_ref,
        out_ref, send_buf, comm_buf, send_sems, recv_sems,
    ):
        my = lax.axis_index("i")

        barrier = pltpu.get_barrier_semaphore()
        for off in range(1, N_DEV):
            peer = lax.rem(my + off, N_DEV)
            pl.semaphore_signal(
                barrier, inc=1,
                device_id=(peer,), device_id_type=pl.DeviceIdType.MESH,
            )
        pl.semaphore_wait(barrier, N_DEV - 1)

        wins = [win0_ref, win1_ref, win2_ref]
        wouts = [wout0_ref, wout1_ref, wout2_ref]

        x_val = x_ref[:, :]
        for r in range(ROUNDS):
            h = jnp.maximum(
                jnp.dot(x_val, wins[r][:, :],
                        preferred_element_type=jnp.float32,
                        precision=lax.Precision.HIGHEST),
                0.0,
            )
            p = jnp.dot(h, wouts[r][:, :],
                        preferred_element_type=jnp.float32,
                        precision=lax.Precision.HIGHEST)
            send_buf[r, :, :] = p

            rdmas = []
            for off in range(1, N_DEV):
                peer = lax.rem(my + off, N_DEV)
                rdma = pltpu.make_async_remote_copy(
                    src_ref=send_buf.at[r],
                    dst_ref=comm_buf.at[r, my],
                    send_sem=send_sems.at[r, peer],
                    recv_sem=recv_sems.at[r, my],
                    device_id=(peer,),
                    device_id_type=pl.DeviceIdType.MESH,
                )
                rdma.start()
                rdmas.append(rdma)
            for rdma in rdmas:
                rdma.wait_send()

            acc = p
            for off in range(1, N_DEV):
                src = lax.rem(my + off, N_DEV)
                recv = pltpu.make_async_remote_copy(
                    src_ref=send_buf.at[r],
                    dst_ref=comm_buf.at[r, src],
                    send_sem=send_sems.at[r, src],
                    recv_sem=recv_sems.at[r, src],
                    device_id=(src,),
                    device_id_type=pl.DeviceIdType.MESH,
                )
                recv.wait_recv()
                acc = acc + comm_buf[r, src]
            x_val = acc

        out_ref[:, :] = lax.dynamic_slice(
            x_val, (my * ROWS_PER, 0), (ROWS_PER, D)
        )

    return pl.pallas_call(
        body,
        out_shape=jax.ShapeDtypeStruct((ROWS_PER, D), jnp.float32),
        in_specs=[pl.BlockSpec(memory_space=pltpu.VMEM)] * 7,
        out_specs=pl.BlockSpec(memory_space=pltpu.VMEM),
        scratch_shapes=[
            pltpu.VMEM((ROUNDS, B, D), jnp.float32),
            pltpu.VMEM((ROUNDS, N_DEV, B, D), jnp.float32),
            pltpu.SemaphoreType.DMA((ROUNDS, N_DEV)),
            pltpu.SemaphoreType.DMA((ROUNDS, N_DEV)),
        ],
        compiler_params=pltpu.CompilerParams(collective_id=0),
    )(x, Win0, Wout0, Win1, Wout1, Win2, Wout2)


# baseline (device time: 27888 ns/iter reference)
import jax
import jax.numpy as jnp
from jax import lax
from jax.experimental import pallas as pl
from jax.experimental.pallas import tpu as pltpu

N_DEV = 8
B = 128
D = 128
ROUNDS = 3
ROWS_PER = B // N_DEV


def kernel(x, Win0, Wout0, Win1, Wout1, Win2, Wout2):
    def body(
        x_ref, win0_ref, wout0_ref, win1_ref, wout1_ref, win2_ref, wout2_ref,
        out_ref, send_buf, comm_buf, send_sems, recv_sems,
    ):
        my = lax.axis_index("i")

        barrier = pltpu.get_barrier_semaphore()
        for off in range(1, N_DEV):
            peer = lax.rem(my + off, N_DEV)
            pl.semaphore_signal(
                barrier, inc=1,
                device_id=(peer,), device_id_type=pl.DeviceIdType.MESH,
            )
        pl.semaphore_wait(barrier, N_DEV - 1)

        wins = [win0_ref, win1_ref, win2_ref]
        wouts = [wout0_ref, wout1_ref, wout2_ref]

        x_val = x_ref[:, :]
        for r in range(ROUNDS):
            h = jnp.maximum(
                jnp.dot(x_val, wins[r][:, :],
                        preferred_element_type=jnp.float32,
                        precision=lax.Precision.HIGHEST),
                0.0,
            )
            p = jnp.dot(h, wouts[r][:, :],
                        preferred_element_type=jnp.float32,
                        precision=lax.Precision.HIGHEST)
            send_buf[r, :, :] = p

            rdmas = []
            for off in range(1, N_DEV):
                peer = lax.rem(my + off, N_DEV)
                rdma = pltpu.make_async_remote_copy(
                    src_ref=send_buf.at[r],
                    dst_ref=comm_buf.at[r, my],
                    send_sem=send_sems.at[r, peer],
                    recv_sem=recv_sems.at[r, my],
                    device_id=(peer,),
                    device_id_type=pl.DeviceIdType.MESH,
                )
                rdma.start()
                rdmas.append(rdma)
            for rdma in rdmas:
                rdma.wait_send()

            last = r == ROUNDS - 1
            row0 = my * ROWS_PER
            acc = send_buf[r, pl.ds(row0, ROWS_PER), :] if last else p
            for off in range(1, N_DEV):
                src = lax.rem(my + off, N_DEV)
                recv = pltpu.make_async_remote_copy(
                    src_ref=send_buf.at[r],
                    dst_ref=comm_buf.at[r, src],
                    send_sem=send_sems.at[r, src],
                    recv_sem=recv_sems.at[r, src],
                    device_id=(src,),
                    device_id_type=pl.DeviceIdType.MESH,
                )
                recv.wait_recv()
                if last:
                    acc = acc + comm_buf[r, src, pl.ds(row0, ROWS_PER), :]
                else:
                    acc = acc + comm_buf[r, src]
            x_val = acc

        out_ref[:, :] = x_val

    return pl.pallas_call(
        body,
        out_shape=jax.ShapeDtypeStruct((ROWS_PER, D), jnp.float32),
        in_specs=[pl.BlockSpec(memory_space=pltpu.VMEM)] * 7,
        out_specs=pl.BlockSpec(memory_space=pltpu.VMEM),
        scratch_shapes=[
            pltpu.VMEM((ROUNDS, B, D), jnp.float32),
            pltpu.VMEM((ROUNDS, N_DEV, B, D), jnp.float32),
            pltpu.SemaphoreType.DMA((ROUNDS, N_DEV)),
            pltpu.SemaphoreType.DMA((ROUNDS, N_DEV)),
        ],
        compiler_params=pltpu.CompilerParams(collective_id=0),
    )(x, Win0, Wout0, Win1, Wout1, Win2, Wout2)


# device time: 26420 ns/iter; 1.0556x vs baseline; 1.0556x over previous
import jax
import jax.numpy as jnp
from jax import lax
from jax.experimental import pallas as pl
from jax.experimental.pallas import tpu as pltpu

N_DEV = 8
B = 128
D = 128
H = 256
ROUNDS = 3
ROWS_PER = B // N_DEV


def kernel(x, Win0, Wout0, Win1, Wout1, Win2, Wout2):
    def body(
        x_hbm, win0_hbm, wout0_hbm, win1_hbm, wout1_hbm, win2_hbm, wout2_hbm,
        out_ref, x_vmem, win_vmem, wout_vmem, send_buf, comm_buf, rs_buf,
        local_sems, send_sems, recv_sems,
    ):
        my = lax.axis_index("i")

        cx = pltpu.make_async_copy(x_hbm, x_vmem, local_sems.at[0])
        cx.start()
        win_hbm = [win0_hbm, win1_hbm, win2_hbm]
        wout_hbm = [wout0_hbm, wout1_hbm, wout2_hbm]
        cwin, cwout = [], []
        for k in range(ROUNDS):
            c = pltpu.make_async_copy(win_hbm[k], win_vmem.at[k],
                                      local_sems.at[1 + k])
            c.start()
            cwin.append(c)
            c = pltpu.make_async_copy(wout_hbm[k], wout_vmem.at[k],
                                      local_sems.at[4 + k])
            c.start()
            cwout.append(c)

        barrier = pltpu.get_barrier_semaphore()
        for off in range(1, N_DEV):
            peer = lax.rem(my + off, N_DEV)
            pl.semaphore_signal(
                barrier, inc=1,
                device_id=(peer,), device_id_type=pl.DeviceIdType.MESH,
            )

        cx.wait()
        x_val = x_vmem[:, :]
        for r in range(ROUNDS):
            last = r == ROUNDS - 1
            cwin[r].wait()
            cwout[r].wait()
            h = jnp.maximum(
                jnp.dot(x_val, win_vmem[r],
                        preferred_element_type=jnp.float32),
                0.0,
            )
            p = jnp.dot(h, wout_vmem[r],
                        preferred_element_type=jnp.float32)
            send_buf[r, :, :] = p

            if r == 0:
                pl.semaphore_wait(barrier, N_DEV - 1)

            rdmas = []
            for off in range(1, N_DEV):
                peer = lax.rem(my + off, N_DEV)
                if last:
                    rdma = pltpu.make_async_remote_copy(
                        src_ref=send_buf.at[r, pl.ds(peer * ROWS_PER, ROWS_PER)],
                        dst_ref=rs_buf.at[my],
                        send_sem=send_sems.at[r, peer],
                        recv_sem=recv_sems.at[r, my],
                        device_id=(peer,),
                        device_id_type=pl.DeviceIdType.MESH,
                    )
                else:
                    rdma = pltpu.make_async_remote_copy(
                        src_ref=send_buf.at[r],
                        dst_ref=comm_buf.at[r, my],
                        send_sem=send_sems.at[r, peer],
                        recv_sem=recv_sems.at[r, my],
                        device_id=(peer,),
                        device_id_type=pl.DeviceIdType.MESH,
                    )
                rdma.start()
                rdmas.append(rdma)
            for rdma in rdmas:
                rdma.wait_send()

            row0 = my * ROWS_PER
            acc = send_buf[r, pl.ds(row0, ROWS_PER), :] if last else p
            for off in range(1, N_DEV):
                src = lax.rem(my + off, N_DEV)
                if last:
                    dst_region = rs_buf.at[src]
                    dummy_src = send_buf.at[r, pl.ds(0, ROWS_PER)]
                else:
                    dst_region = comm_buf.at[r, src]
                    dummy_src = send_buf.at[r]
                recv = pltpu.make_async_remote_copy(
                    src_ref=dummy_src,
                    dst_ref=dst_region,
                    send_sem=send_sems.at[r, src],
                    recv_sem=recv_sems.at[r, src],
                    device_id=(src,),
                    device_id_type=pl.DeviceIdType.MESH,
                )
                recv.wait_recv()
                if last:
                    acc = acc + rs_buf[src]
                else:
                    acc = acc + comm_buf[r, src]
            x_val = acc

        out_ref[:, :] = x_val

    return pl.pallas_call(
        body,
        out_shape=jax.ShapeDtypeStruct((ROWS_PER, D), jnp.float32),
        in_specs=[pl.BlockSpec(memory_space=pl.ANY)] * 7,
        out_specs=pl.BlockSpec(memory_space=pltpu.VMEM),
        scratch_shapes=[
            pltpu.VMEM((B, D), jnp.float32),
            pltpu.VMEM((ROUNDS, D, H), jnp.float32),
            pltpu.VMEM((ROUNDS, H, D), jnp.float32),
            pltpu.VMEM((ROUNDS, B, D), jnp.float32),
            pltpu.VMEM((ROUNDS - 1, N_DEV, B, D), jnp.float32),
            pltpu.VMEM((N_DEV, ROWS_PER, D), jnp.float32),
            pltpu.SemaphoreType.DMA((7,)),
            pltpu.SemaphoreType.DMA((ROUNDS, N_DEV)),
            pltpu.SemaphoreType.DMA((ROUNDS, N_DEV)),
        ],
        compiler_params=pltpu.CompilerParams(collective_id=0),
    )(x, Win0, Wout0, Win1, Wout1, Win2, Wout2)


# device time: 22065 ns/iter; 1.2639x vs baseline; 1.1974x over previous
import jax
import jax.numpy as jnp
from jax import lax
from jax.experimental import pallas as pl
from jax.experimental.pallas import tpu as pltpu

N_DEV = 8
B = 128
D = 128
H = 256
ROUNDS = 3
ROWS_PER = B // N_DEV


def kernel(x, Win0, Wout0, Win1, Wout1, Win2, Wout2):
    def body(
        x_hbm, wins_hbm, wouts_hbm,
        out_ref, x_vmem, win_vmem, wout_vmem, send_buf, comm_buf, rs_buf,
        local_sems, send_sems, recv_sems,
    ):
        my = lax.axis_index("i")

        cx = pltpu.make_async_copy(x_hbm, x_vmem, local_sems.at[0])
        cx.start()
        cwin, cwout = [], []
        for k in range(ROUNDS):
            c = pltpu.make_async_copy(wins_hbm.at[k], win_vmem.at[k],
                                      local_sems.at[1 + k])
            c.start()
            cwin.append(c)
            c = pltpu.make_async_copy(wouts_hbm.at[k], wout_vmem.at[k],
                                      local_sems.at[4 + k])
            c.start()
            cwout.append(c)

        barrier = pltpu.get_barrier_semaphore()
        for off in range(1, N_DEV):
            peer = lax.rem(my + off, N_DEV)
            pl.semaphore_signal(
                barrier, inc=1,
                device_id=(peer,), device_id_type=pl.DeviceIdType.MESH,
            )

        cx.wait()
        x_val = x_vmem[:, :]
        for r in range(ROUNDS):
            last = r == ROUNDS - 1
            cwin[r].wait()
            cwout[r].wait()
            h = jnp.maximum(
                jnp.dot(x_val, win_vmem[r],
                        preferred_element_type=jnp.float32),
                0.0,
            )
            p = jnp.dot(h, wout_vmem[r],
                        preferred_element_type=jnp.float32)
            send_buf[r, :, :] = p

            if r == 0:
                pl.semaphore_wait(barrier, N_DEV - 1)

            rdmas = []
            for off in range(1, N_DEV):
                peer = lax.rem(my + off, N_DEV)
                if last:
                    rdma = pltpu.make_async_remote_copy(
                        src_ref=send_buf.at[r, pl.ds(peer * ROWS_PER, ROWS_PER)],
                        dst_ref=rs_buf.at[my],
                        send_sem=send_sems.at[r, peer],
                        recv_sem=recv_sems.at[r, my],
                        device_id=(peer,),
                        device_id_type=pl.DeviceIdType.MESH,
                    )
                else:
                    rdma = pltpu.make_async_remote_copy(
                        src_ref=send_buf.at[r],
                        dst_ref=comm_buf.at[r, my],
                        send_sem=send_sems.at[r, peer],
                        recv_sem=recv_sems.at[r, my],
                        device_id=(peer,),
                        device_id_type=pl.DeviceIdType.MESH,
                    )
                rdma.start()
                rdmas.append(rdma)
            for rdma in rdmas:
                rdma.wait_send()

            row0 = my * ROWS_PER
            acc = send_buf[r, pl.ds(row0, ROWS_PER), :] if last else p
            for off in range(1, N_DEV):
                src = lax.rem(my + off, N_DEV)
                if last:
                    dst_region = rs_buf.at[src]
                    dummy_src = send_buf.at[r, pl.ds(0, ROWS_PER)]
                else:
                    dst_region = comm_buf.at[r, src]
                    dummy_src = send_buf.at[r]
                recv = pltpu.make_async_remote_copy(
                    src_ref=dummy_src,
                    dst_ref=dst_region,
                    send_sem=send_sems.at[r, src],
                    recv_sem=recv_sems.at[r, src],
                    device_id=(src,),
                    device_id_type=pl.DeviceIdType.MESH,
                )
                recv.wait_recv()
                if last:
                    acc = acc + rs_buf[src]
                else:
                    acc = acc + comm_buf[r, src]
            x_val = acc

        out_ref[:, :] = x_val

    return pl.pallas_call(
        body,
        out_shape=jax.ShapeDtypeStruct((ROWS_PER, D), jnp.float32),
        in_specs=[pl.BlockSpec(memory_space=pl.ANY)] * 3,
        out_specs=pl.BlockSpec(memory_space=pltpu.VMEM),
        scratch_shapes=[
            pltpu.VMEM((B, D), jnp.float32),
            pltpu.VMEM((ROUNDS, D, H), jnp.float32),
            pltpu.VMEM((ROUNDS, H, D), jnp.float32),
            pltpu.VMEM((ROUNDS, B, D), jnp.float32),
            pltpu.VMEM((ROUNDS - 1, N_DEV, B, D), jnp.float32),
            pltpu.VMEM((N_DEV, ROWS_PER, D), jnp.float32),
            pltpu.SemaphoreType.DMA((7,)),
            pltpu.SemaphoreType.DMA((ROUNDS, N_DEV)),
            pltpu.SemaphoreType.DMA((ROUNDS, N_DEV)),
        ],
        compiler_params=pltpu.CompilerParams(collective_id=0),
    )(
        x,
        jnp.stack([Win0, Win1, Win2]),
        jnp.stack([Wout0, Wout1, Wout2]),
    )


# device time: 18447 ns/iter; 1.5118x vs baseline; 1.1961x over previous
import jax
import jax.numpy as jnp
from jax import lax
from jax.experimental import pallas as pl
from jax.experimental.pallas import tpu as pltpu

N_DEV = 8
B = 128
D = 128
H = 256
ROUNDS = 3
ROWS_PER = B // N_DEV


def kernel(x, Win0, Wout0, Win1, Wout1, Win2, Wout2):
    def body(
        wins_hbm, wouts_hbm,
        out_ref, x_vmem, win_vmem, wout_vmem, send_buf, comm_buf, rs_buf,
        local_sems, send_sems, recv_sems,
    ):
        my = lax.axis_index("i")

        cx = pltpu.make_async_copy(
            wins_hbm.at[ROUNDS, :, pl.ds(0, D)], x_vmem, local_sems.at[0]
        )
        cx.start()
        cwin, cwout = [], []
        for k in range(ROUNDS):
            c = pltpu.make_async_copy(wins_hbm.at[k], win_vmem.at[k],
                                      local_sems.at[1 + k])
            c.start()
            cwin.append(c)
            c = pltpu.make_async_copy(wouts_hbm.at[k], wout_vmem.at[k],
                                      local_sems.at[4 + k])
            c.start()
            cwout.append(c)

        barrier = pltpu.get_barrier_semaphore()
        for off in range(1, N_DEV):
            peer = lax.rem(my + off, N_DEV)
            pl.semaphore_signal(
                barrier, inc=1,
                device_id=(peer,), device_id_type=pl.DeviceIdType.MESH,
            )

        cx.wait()
        x_val = x_vmem[:, :]
        for r in range(ROUNDS):
            last = r == ROUNDS - 1
            cwin[r].wait()
            cwout[r].wait()
            h = jnp.maximum(
                jnp.dot(x_val, win_vmem[r],
                        preferred_element_type=jnp.float32),
                0.0,
            )
            p = jnp.dot(h, wout_vmem[r],
                        preferred_element_type=jnp.float32)
            send_buf[r, :, :] = p.astype(jnp.bfloat16)

            if r == 0:
                pl.semaphore_wait(barrier, N_DEV - 1)

            rdmas = []
            for off in range(1, N_DEV):
                peer = lax.rem(my + off, N_DEV)
                if last:
                    rdma = pltpu.make_async_remote_copy(
                        src_ref=send_buf.at[r, pl.ds(peer * ROWS_PER, ROWS_PER)],
                        dst_ref=rs_buf.at[my],
                        send_sem=send_sems.at[r, peer],
                        recv_sem=recv_sems.at[r, my],
                        device_id=(peer,),
                        device_id_type=pl.DeviceIdType.MESH,
                    )
                else:
                    rdma = pltpu.make_async_remote_copy(
                        src_ref=send_buf.at[r],
                        dst_ref=comm_buf.at[r, my],
                        send_sem=send_sems.at[r, peer],
                        recv_sem=recv_sems.at[r, my],
                        device_id=(peer,),
                        device_id_type=pl.DeviceIdType.MESH,
                    )
                rdma.start()
                rdmas.append(rdma)
            for rdma in rdmas:
                rdma.wait_send()

            row0 = my * ROWS_PER
            if last:
                acc = send_buf[r, pl.ds(row0, ROWS_PER), :].astype(jnp.float32)
            else:
                acc = p
            for off in range(1, N_DEV):
                src = lax.rem(my + off, N_DEV)
                if last:
                    dst_region = rs_buf.at[src]
                    dummy_src = send_buf.at[r, pl.ds(0, ROWS_PER)]
                else:
                    dst_region = comm_buf.at[r, src]
                    dummy_src = send_buf.at[r]
                recv = pltpu.make_async_remote_copy(
                    src_ref=dummy_src,
                    dst_ref=dst_region,
                    send_sem=send_sems.at[r, src],
                    recv_sem=recv_sems.at[r, src],
                    device_id=(src,),
                    device_id_type=pl.DeviceIdType.MESH,
                )
                recv.wait_recv()
                if last:
                    acc = acc + rs_buf[src].astype(jnp.float32)
                else:
                    acc = acc + comm_buf[r, src].astype(jnp.float32)
            x_val = acc

        out_ref[:, :] = x_val

    return pl.pallas_call(
        body,
        out_shape=jax.ShapeDtypeStruct((ROWS_PER, D), jnp.float32),
        in_specs=[pl.BlockSpec(memory_space=pl.ANY)] * 2,
        out_specs=pl.BlockSpec(memory_space=pltpu.VMEM),
        scratch_shapes=[
            pltpu.VMEM((B, D), jnp.float32),
            pltpu.VMEM((ROUNDS, D, H), jnp.float32),
            pltpu.VMEM((ROUNDS, H, D), jnp.float32),
            pltpu.VMEM((ROUNDS, B, D), jnp.bfloat16),
            pltpu.VMEM((ROUNDS - 1, N_DEV, B, D), jnp.bfloat16),
            pltpu.VMEM((N_DEV, ROWS_PER, D), jnp.bfloat16),
            pltpu.SemaphoreType.DMA((7,)),
            pltpu.SemaphoreType.DMA((ROUNDS, N_DEV)),
            pltpu.SemaphoreType.DMA((ROUNDS, N_DEV)),
        ],
        compiler_params=pltpu.CompilerParams(collective_id=0),
    )(
        jnp.stack([Win0, Win1, Win2,
                   jnp.pad(x, ((0, 0), (0, H - D)))]),
        jnp.stack([Wout0, Wout1, Wout2]),
    )
